# Initial kernel scaffold; baseline (speedup 1.0000x reference)
import jax
import jax.numpy as jnp
from jax.experimental import pallas as pl


def kernel(x, adj_indices, adj_values, W, b):
    raise NotImplementedError("write your pallas kernel here")


# speedup vs baseline: 4.3618x; 4.3618x over previous
import functools

import jax
import jax.numpy as jnp
from jax import lax
from jax.experimental import pallas as pl
from jax.experimental.pallas import tpu as pltpu
from jax.experimental.pallas import tpu_sc as plsc

_N = 10000
_E = 320000
_D = 128

_NC = 2
_NS = 16
_CHUNK = 80
_EDGES_PER_TILE = _E // (_NC * _NS)
_NCHUNKS = _EDGES_PER_TILE // _CHUNK
_ZROWS = 200
_NZCHUNKS = _N // _ZROWS
_BN = 2000


def _mm_body(x_ref, w_ref, b_ref, y_ref):
    y_ref[...] = lax.dot_general(
        x_ref[...], w_ref[...],
        (((1,), (1,)), ((), ())),
        preferred_element_type=jnp.float32,
    ) + b_ref[...]


def _add_body(a_ref, b_ref, o_ref):
    o_ref[...] = a_ref[...] + b_ref[...]


def _sc_body(y_hbm, row_hbm, col_hbm, val_hbm, out_hbm,
             colbuf, rowbuf, valbuf, rows, zbuf, acc, sem):
    c = lax.axis_index("c")
    s = lax.axis_index("s")

    def zfill(i, carry):
        r = i // 8
        g = (i % 8) * 16
        zbuf[r, pl.ds(g, 16)] = jnp.zeros((16,), jnp.float32)
        return carry
    lax.fori_loop(0, _ZROWS * 8, zfill, 0)

    for k in range(_NZCHUNKS // _NS + 1):
        idx = s + _NS * k

        @pl.when(idx < _NZCHUNKS)
        def _():
            pltpu.sync_copy(zbuf, acc.at[pl.ds(idx * _ZROWS, _ZROWS)])

    plsc.subcore_barrier()

    base = (c * _NS + s) * _EDGES_PER_TILE

    def chunk_body(j, carry):
        eb = base + j * _CHUNK
        pltpu.sync_copy(col_hbm.at[pl.ds(eb, _CHUNK)], colbuf)
        pltpu.sync_copy(row_hbm.at[pl.ds(eb, _CHUNK)], rowbuf)
        pltpu.sync_copy(val_hbm.at[pl.ds(eb, _CHUNK)], valbuf)
        pltpu.async_copy(y_hbm.at[colbuf], rows, sem).wait()

        def group_body(gi, cc):
            vv = valbuf[pl.ds(gi * 16, 16)]
            for r in range(16):
                v = vv[r]
                e = gi * 16 + r
                for g in range(8):
                    sl = pl.ds(g * 16, 16)
                    rows[e, sl] = rows[e, sl] * v
            return cc
        lax.fori_loop(0, _CHUNK // 16, group_body, 0)

        pltpu.sync_copy(rows, acc.at[rowbuf], add=True)
        return carry
    lax.fori_loop(0, _NCHUNKS, chunk_body, 0)

    plsc.subcore_barrier()

    for k in range(_NZCHUNKS // _NS + 1):
        idx = s + _NS * k

        @pl.when(idx < _NZCHUNKS)
        def _():
            pltpu.sync_copy(acc.at[pl.ds(idx * _ZROWS, _ZROWS)], zbuf)
            pltpu.sync_copy(zbuf, out_hbm.at[pl.ds(c * _N + idx * _ZROWS, _ZROWS)])


_sc_spmm = pl.kernel(
    _sc_body,
    out_type=jax.ShapeDtypeStruct((_NC * _N, _D), jnp.float32),
    mesh=plsc.VectorSubcoreMesh(core_axis_name="c", subcore_axis_name="s"),
    scratch_types=[
        pltpu.VMEM((_CHUNK,), jnp.int32),
        pltpu.VMEM((_CHUNK,), jnp.int32),
        pltpu.VMEM((_CHUNK,), jnp.float32),
        pltpu.VMEM((_CHUNK, _D), jnp.float32),
        pltpu.VMEM((_ZROWS, _D), jnp.float32),
        pltpu.VMEM_SHARED((_N, _D), jnp.float32),
        pltpu.SemaphoreType.DMA,
    ],
)

_matmul = pl.pallas_call(
    _mm_body,
    grid=(_N // _BN,),
    in_specs=[
        pl.BlockSpec((_BN, _D), lambda i: (i, 0)),
        pl.BlockSpec((_D, _D), lambda i: (0, 0)),
        pl.BlockSpec((1, _D), lambda i: (0, 0)),
    ],
    out_specs=pl.BlockSpec((_BN, _D), lambda i: (i, 0)),
    out_shape=jax.ShapeDtypeStruct((_N, _D), jnp.float32),
)

_padd = pl.pallas_call(
    _add_body,
    grid=(_N // _BN,),
    in_specs=[
        pl.BlockSpec((_BN, _D), lambda i: (i, 0)),
        pl.BlockSpec((_BN, _D), lambda i: (i, 0)),
    ],
    out_specs=pl.BlockSpec((_BN, _D), lambda i: (i, 0)),
    out_shape=jax.ShapeDtypeStruct((_N, _D), jnp.float32),
)


@jax.jit
def kernel(x, adj_indices, adj_values, W, b):
    y = _matmul(x, W, b.reshape(1, _D))
    row = adj_indices[0]
    col = adj_indices[1]
    partials = _sc_spmm(y, row, col, adj_values)
    return _padd(partials[:_N], partials[_N:])


# speedup vs baseline: 11.7026x; 2.6830x over previous
import jax
import jax.numpy as jnp
from jax import lax
from jax.experimental import pallas as pl
from jax.experimental.pallas import tpu as pltpu
from jax.experimental.pallas import tpu_sc as plsc

_N = 10000
_E = 320000
_D = 128

_NC = 2
_NS = 16
_CHUNK = 80
_EPT = _E // (_NC * _NS)
_NCHUNKS = _EPT // _CHUNK
_NZ = _N // _CHUNK
_BN = 2000


def _mm_body(x_ref, w_ref, b_ref, y_ref):
    y_ref[...] = lax.dot_general(
        x_ref[...], w_ref[...],
        (((1,), (1,)), ((), ())),
        preferred_element_type=jnp.float32,
    ) + b_ref[...]


def _add_body(a_ref, b_ref, o_ref):
    o_ref[...] = a_ref[...] + b_ref[...]


def _sc_body(y_hbm, row_hbm, col_hbm, val_hbm, out_hbm,
             colall, rows0, rows1, rows2, rb0, rb1, rb2, vb0, vb1, vb2, acc,
             g0, g1, g2, s0, s1, s2):
    c = lax.axis_index("c")
    s = lax.axis_index("s")
    rows = (rows0, rows1, rows2)
    rbs = (rb0, rb1, rb2)
    vbs = (vb0, vb1, vb2)
    gsem = (g0, g1, g2)
    ssem = (s0, s1, s2)

    def zfill(i, carry):
        r = i // 8
        g = (i % 8) * 16
        rows0[r, pl.ds(g, 16)] = jnp.zeros((16,), jnp.float32)
        return carry
    lax.fori_loop(0, _CHUNK * 8, zfill, 0)

    for k in range(_NZ // _NS + 1):
        idx = s + _NS * k

        @pl.when(idx < _NZ)
        def _():
            pltpu.sync_copy(rows0, acc.at[pl.ds(idx * _CHUNK, _CHUNK)])

    plsc.subcore_barrier()

    base = (c * _NS + s) * _EPT
    pltpu.sync_copy(col_hbm.at[pl.ds(base, _EPT)], colall)

    def issue(k, b):
        pltpu.async_copy(
            y_hbm.at[colall.at[pl.ds(k * _CHUNK, _CHUNK)]], rows[b], gsem[b])
        pltpu.async_copy(
            row_hbm.at[pl.ds(base + k * _CHUNK, _CHUNK)], rbs[b], gsem[b])
        pltpu.async_copy(
            val_hbm.at[pl.ds(base + k * _CHUNK, _CHUNK)], vbs[b], gsem[b])

    def wait_issue(b):
        pltpu.make_async_copy(
            y_hbm.at[colall.at[pl.ds(0, _CHUNK)]], rows[b], gsem[b]).wait()
        pltpu.make_async_copy(
            row_hbm.at[pl.ds(0, _CHUNK)], rbs[b], gsem[b]).wait()
        pltpu.make_async_copy(
            val_hbm.at[pl.ds(0, _CHUNK)], vbs[b], gsem[b]).wait()

    def issue_scatter(b):
        pltpu.async_copy(rows[b], acc.at[rbs[b]], ssem[b], add=True)

    def wait_scatter(b):
        pltpu.make_async_copy(rows[b], acc.at[rbs[b]], ssem[b]).wait()

    issue(0, 0)
    issue(1, 1)

    def outer(g, carry):
        for b in range(3):
            j = g * 3 + b
            bn = (b + 2) % 3

            @pl.when(j < _NCHUNKS)
            def _():
                wait_issue(b)

                def scale_grp(i, cc):
                    vv = vbs[b][pl.ds(i * 16, 16)]
                    for r in range(16):
                        for gg in range(8):
                            sl = pl.ds(gg * 16, 16)
                            rows[b][i * 16 + r, sl] = (
                                rows[b][i * 16 + r, sl] * vv[r])
                    return cc
                lax.fori_loop(0, _CHUNK // 16, scale_grp, 0)
                issue_scatter(b)

            @pl.when(jnp.logical_and(j >= 1, j <= _NCHUNKS))
            def _():
                wait_scatter(bn)

            @pl.when(j + 2 < _NCHUNKS)
            def _():
                issue(j + 2, bn)

        return carry

    lax.fori_loop(0, (_NCHUNKS + 1 + 2) // 3, outer, 0)

    plsc.subcore_barrier()

    for k in range(_NZ // _NS + 1):
        idx = s + _NS * k

        @pl.when(idx < _NZ)
        def _():
            pltpu.sync_copy(acc.at[pl.ds(idx * _CHUNK, _CHUNK)], rows0)
            pltpu.sync_copy(rows0, out_hbm.at[pl.ds(c * _N + idx * _CHUNK, _CHUNK)])


_sc_spmm = pl.kernel(
    _sc_body,
    out_type=jax.ShapeDtypeStruct((_NC * _N, _D), jnp.float32),
    mesh=plsc.VectorSubcoreMesh(core_axis_name="c", subcore_axis_name="s"),
    scratch_types=[
        pltpu.VMEM((_EPT,), jnp.int32),
        pltpu.VMEM((_CHUNK, _D), jnp.float32),
        pltpu.VMEM((_CHUNK, _D), jnp.float32),
        pltpu.VMEM((_CHUNK, _D), jnp.float32),
        pltpu.VMEM((_CHUNK,), jnp.int32),
        pltpu.VMEM((_CHUNK,), jnp.int32),
        pltpu.VMEM((_CHUNK,), jnp.int32),
        pltpu.VMEM((_CHUNK,), jnp.float32),
        pltpu.VMEM((_CHUNK,), jnp.float32),
        pltpu.VMEM((_CHUNK,), jnp.float32),
        pltpu.VMEM_SHARED((_N, _D), jnp.float32),
        pltpu.SemaphoreType.DMA,
        pltpu.SemaphoreType.DMA,
        pltpu.SemaphoreType.DMA,
        pltpu.SemaphoreType.DMA,
        pltpu.SemaphoreType.DMA,
        pltpu.SemaphoreType.DMA,
    ],
)

_matmul = pl.pallas_call(
    _mm_body,
    grid=(_N // _BN,),
    in_specs=[
        pl.BlockSpec((_BN, _D), lambda i: (i, 0)),
        pl.BlockSpec((_D, _D), lambda i: (0, 0)),
        pl.BlockSpec((1, _D), lambda i: (0, 0)),
    ],
    out_specs=pl.BlockSpec((_BN, _D), lambda i: (i, 0)),
    out_shape=jax.ShapeDtypeStruct((_N, _D), jnp.float32),
)

_padd = pl.pallas_call(
    _add_body,
    grid=(_N // _BN,),
    in_specs=[
        pl.BlockSpec((_BN, _D), lambda i: (i, 0)),
        pl.BlockSpec((_BN, _D), lambda i: (i, 0)),
    ],
    out_specs=pl.BlockSpec((_BN, _D), lambda i: (i, 0)),
    out_shape=jax.ShapeDtypeStruct((_N, _D), jnp.float32),
)


@jax.jit
def kernel(x, adj_indices, adj_values, W, b):
    y = _matmul(x, W, b.reshape(1, _D))
    row = adj_indices[0]
    col = adj_indices[1]
    partials = _sc_spmm(y, row, col, adj_values)
    return _padd(partials[:_N], partials[_N:])


# speedup vs baseline: 11.8730x; 1.0146x over previous
import jax
import jax.numpy as jnp
from jax import lax
from jax.experimental import pallas as pl
from jax.experimental.pallas import tpu as pltpu
from jax.experimental.pallas import tpu_sc as plsc

_N = 10000
_E = 320000
_D = 128

_NC = 2
_NS = 16
_CHUNK = 80
_EPT = _E // (_NC * _NS)
_NCHUNKS = _EPT // _CHUNK
_NZ = _N // _CHUNK
_BN = 2000


def _mm_body(x_ref, w_ref, b_ref, y_ref):
    y_ref[...] = lax.dot_general(
        x_ref[...], w_ref[...],
        (((1,), (1,)), ((), ())),
        preferred_element_type=jnp.float32,
    ) + b_ref[...]


def _add_body(a_ref, b_ref, o_ref):
    o_ref[...] = a_ref[...] + b_ref[...]


def _sc_body(y_hbm, row_hbm, col_hbm, val_hbm, out_hbm,
             colall, rows0, rows1, rows2, rb0, rb1, rb2, vb0, vb1, vb2, acc,
             g0, g1, g2, s0, s1, s2):
    c = lax.axis_index("c")
    s = lax.axis_index("s")
    rows = (rows0, rows1, rows2)
    rbs = (rb0, rb1, rb2)
    vbs = (vb0, vb1, vb2)
    gsem = (g0, g1, g2)
    ssem = (s0, s1, s2)

    base = (c * _NS + s) * _EPT
    pltpu.sync_copy(col_hbm.at[pl.ds(base, _EPT)], colall)

    def issue(k, b):
        pltpu.async_copy(
            y_hbm.at[colall.at[pl.ds(k * _CHUNK, _CHUNK)]], rows[b], gsem[b])
        pltpu.async_copy(
            row_hbm.at[pl.ds(base + k * _CHUNK, _CHUNK)], rbs[b], gsem[b])
        pltpu.async_copy(
            val_hbm.at[pl.ds(base + k * _CHUNK, _CHUNK)], vbs[b], gsem[b])

    def wait_issue(b):
        pltpu.make_async_copy(
            y_hbm.at[colall.at[pl.ds(0, _CHUNK)]], rows[b], gsem[b]).wait()
        pltpu.make_async_copy(
            row_hbm.at[pl.ds(0, _CHUNK)], rbs[b], gsem[b]).wait()
        pltpu.make_async_copy(
            val_hbm.at[pl.ds(0, _CHUNK)], vbs[b], gsem[b]).wait()

    def issue_scatter(b):
        pltpu.async_copy(rows[b], acc.at[rbs[b]], ssem[b], add=True)

    def wait_scatter(b):
        pltpu.make_async_copy(rows[b], acc.at[rbs[b]], ssem[b]).wait()

    issue(0, 0)
    issue(1, 1)

    def zfill(i, carry):
        r = i // 8
        g = (i % 8) * 16
        rows2[r, pl.ds(g, 16)] = jnp.zeros((16,), jnp.float32)
        return carry
    lax.fori_loop(0, _CHUNK * 8, zfill, 0)

    for k in range(_NZ // _NS + 1):
        idx = s + _NS * k

        @pl.when(idx < _NZ)
        def _():
            pltpu.sync_copy(rows2, acc.at[pl.ds(idx * _CHUNK, _CHUNK)])

    plsc.subcore_barrier()

    def outer(g, carry):
        for b in range(3):
            j = g * 3 + b
            bn = (b + 2) % 3

            @pl.when(j < _NCHUNKS)
            def _():
                wait_issue(b)

                def scale_grp(i, cc):
                    vv = vbs[b][pl.ds(i * 16, 16)]
                    for r in range(16):
                        for gg in range(8):
                            sl = pl.ds(gg * 16, 16)
                            rows[b][i * 16 + r, sl] = (
                                rows[b][i * 16 + r, sl] * vv[r])
                    return cc
                lax.fori_loop(0, _CHUNK // 16, scale_grp, 0)
                issue_scatter(b)

            @pl.when(jnp.logical_and(j >= 1, j <= _NCHUNKS))
            def _():
                wait_scatter(bn)

            @pl.when(j + 2 < _NCHUNKS)
            def _():
                issue(j + 2, bn)

        return carry

    lax.fori_loop(0, (_NCHUNKS + 1 + 2) // 3, outer, 0)

    plsc.subcore_barrier()

    for k in range(_NZ // _NS + 1):
        idx = s + _NS * k

        @pl.when(idx < _NZ)
        def _():
            pltpu.sync_copy(acc.at[pl.ds(idx * _CHUNK, _CHUNK)],
                            out_hbm.at[pl.ds(c * _N + idx * _CHUNK, _CHUNK)])


_sc_spmm = pl.kernel(
    _sc_body,
    out_type=jax.ShapeDtypeStruct((_NC * _N, _D), jnp.float32),
    mesh=plsc.VectorSubcoreMesh(core_axis_name="c", subcore_axis_name="s"),
    scratch_types=[
        pltpu.VMEM((_EPT,), jnp.int32),
        pltpu.VMEM((_CHUNK, _D), jnp.float32),
        pltpu.VMEM((_CHUNK, _D), jnp.float32),
        pltpu.VMEM((_CHUNK, _D), jnp.float32),
        pltpu.VMEM((_CHUNK,), jnp.int32),
        pltpu.VMEM((_CHUNK,), jnp.int32),
        pltpu.VMEM((_CHUNK,), jnp.int32),
        pltpu.VMEM((_CHUNK,), jnp.float32),
        pltpu.VMEM((_CHUNK,), jnp.float32),
        pltpu.VMEM((_CHUNK,), jnp.float32),
        pltpu.VMEM_SHARED((_N, _D), jnp.float32),
        pltpu.SemaphoreType.DMA,
        pltpu.SemaphoreType.DMA,
        pltpu.SemaphoreType.DMA,
        pltpu.SemaphoreType.DMA,
        pltpu.SemaphoreType.DMA,
        pltpu.SemaphoreType.DMA,
    ],
)

_matmul = pl.pallas_call(
    _mm_body,
    grid=(_N // _BN,),
    in_specs=[
        pl.BlockSpec((_BN, _D), lambda i: (i, 0)),
        pl.BlockSpec((_D, _D), lambda i: (0, 0)),
        pl.BlockSpec((1, _D), lambda i: (0, 0)),
    ],
    out_specs=pl.BlockSpec((_BN, _D), lambda i: (i, 0)),
    out_shape=jax.ShapeDtypeStruct((_N, _D), jnp.float32),
)

_padd = pl.pallas_call(
    _add_body,
    grid=(_N // _BN,),
    in_specs=[
        pl.BlockSpec((_BN, _D), lambda i: (i, 0)),
        pl.BlockSpec((_BN, _D), lambda i: (i, 0)),
    ],
    out_specs=pl.BlockSpec((_BN, _D), lambda i: (i, 0)),
    out_shape=jax.ShapeDtypeStruct((_N, _D), jnp.float32),
)


@jax.jit
def kernel(x, adj_indices, adj_values, W, b):
    y = _matmul(x, W, b.reshape(1, _D))
    row = adj_indices[0]
    col = adj_indices[1]
    partials = _sc_spmm(y, row, col, adj_values)
    return _padd(partials[:_N], partials[_N:])
